# baseline (speedup 1.0000x reference)
import functools

import jax
import jax.numpy as jnp
from jax import lax
from jax.experimental import pallas as pl
from jax.experimental.pallas import tpu as pltpu
from jax.experimental.pallas import tpu_sc as plsc

EMB_DIM = 64
NC = 2
NS = 16
NW = NC * NS
CHUNK = 128
NBUF = 2

_PB = 4096
_PH = _PB // 2


def _pack_rows(table_t):
    n = table_t.shape[1]
    grid = (n + _PB - 1) // _PB

    def body(x_ref, o_ref):
        x = x_ref[...]
        eye = (lax.broadcasted_iota(jnp.int32, (EMB_DIM, EMB_DIM), 0)
               == lax.broadcasted_iota(jnp.int32, (EMB_DIM, EMB_DIM), 1)
               ).astype(jnp.float32)
        xh = x.astype(jnp.bfloat16).astype(jnp.float32)
        xl = x - xh
        dims = (((0,), (0,)), ((), ()))
        y = (lax.dot_general(xh, eye, dims, preferred_element_type=jnp.float32)
             + lax.dot_general(xl, eye, dims,
                               preferred_element_type=jnp.float32))
        o_ref[...] = jnp.concatenate([y[:_PH], y[_PH:]], axis=1)

    return pl.pallas_call(
        body,
        grid=(grid,),
        in_specs=[pl.BlockSpec((EMB_DIM, _PB), lambda i: (0, i))],
        out_specs=pl.BlockSpec((_PH, 2 * EMB_DIM), lambda i: (i, 0)),
        out_shape=jax.ShapeDtypeStruct((grid * _PH, 2 * EMB_DIM),
                                       jnp.float32),
    )(table_t)


@functools.lru_cache(maxsize=None)
def _make_sc_scores(P: int):
    PW = P // NW
    NCHUNK = PW // CHUNK

    mesh = plsc.VectorSubcoreMesh(
        core_axis_name="c", subcore_axis_name="s",
        num_cores=NC, num_subcores=NS,
    )

    @functools.partial(
        pl.kernel,
        mesh=mesh,
        out_type=jax.ShapeDtypeStruct((P,), jnp.float32),
        scratch_types=[
            pltpu.VMEM((NCHUNK, CHUNK), jnp.int32),
            pltpu.VMEM((NCHUNK, CHUNK), jnp.int32),
            pltpu.VMEM((NCHUNK, CHUNK), jnp.int32),
            pltpu.VMEM((NCHUNK, CHUNK), jnp.int32),
            pltpu.VMEM((NBUF, CHUNK, 2 * EMB_DIM), jnp.float32),
            pltpu.VMEM((NBUF, CHUNK, 2 * EMB_DIM), jnp.float32),
            pltpu.VMEM((PW,), jnp.float32),
            pltpu.SemaphoreType.DMA,
            pltpu.SemaphoreType.DMA,
            pltpu.SemaphoreType.DMA,
            pltpu.SemaphoreType.DMA,
        ],
    )
    def sc_scores(u_hbm, v_hbm, iug_hbm, ivg_hbm, iuo_hbm, ivo_hbm, out_hbm,
                  iug_v, ivg_v, iuo_v, ivo_v, ubuf, vbuf, sv,
                  su0, su1, sv0, sv1):
        sems_u = [su0, su1]
        sems_v = [sv0, sv1]
        wid = lax.axis_index("s") * NC + lax.axis_index("c")

        pltpu.sync_copy(iug_hbm.at[wid], iug_v)
        pltpu.sync_copy(ivg_hbm.at[wid], ivg_v)
        pltpu.sync_copy(iuo_hbm.at[wid], iuo_v)
        pltpu.sync_copy(ivo_hbm.at[wid], ivo_v)

        def start(g, slot):
            pltpu.async_copy(u_hbm.at[iug_v.at[g]], ubuf.at[slot], sems_u[slot])
            pltpu.async_copy(v_hbm.at[ivg_v.at[g]], vbuf.at[slot], sems_v[slot])

        def wait(g, slot):
            pltpu.make_async_copy(
                u_hbm.at[iug_v.at[g]], ubuf.at[slot], sems_u[slot]).wait()
            pltpu.make_async_copy(
                v_hbm.at[ivg_v.at[g]], vbuf.at[slot], sems_v[slot]).wait()

        lanes = lax.iota(jnp.int32, 16)

        def perm(x, idx):
            return lax.gather(
                x, idx[:, None],
                lax.GatherDimensionNumbers(
                    offset_dims=(), collapsed_slice_dims=(0,),
                    start_index_map=(0,)),
                slice_sizes=(1,),
                mode=lax.GatherScatterMode.PROMISE_IN_BOUNDS)

        def compute(g, slot):
            ub = ubuf.at[slot]
            vb = vbuf.at[slot]

            def body(j, _):
                uoff16 = iuo_v[g, pl.ds(j * 16, 16)]
                voff16 = ivo_v[g, pl.ds(j * 16, 16)]
                acc = jnp.zeros((16,), jnp.float32)
                for t in range(16):
                    p = j * 16 + t
                    uo = pl.multiple_of(uoff16[t], EMB_DIM)
                    vo = pl.multiple_of(voff16[t], EMB_DIM)
                    d = jnp.zeros((16,), jnp.float32)
                    for q in range(EMB_DIM // 16):
                        d = d + (ub[p, pl.ds(uo + q * 16, 16)]
                                 * vb[p, pl.ds(vo + q * 16, 16)])
                    for s_ in (8, 4, 2, 1):
                        d = d + perm(d, lanes ^ s_)
                    acc = jnp.where(lanes == t, d, acc)
                sv[pl.ds(g * CHUNK + j * 16, 16)] = acc
                return 0

            lax.fori_loop(0, CHUNK // 16, body, 0)

        for b in range(NBUF):
            start(b, b)

        def outer(gg, _):
            for b in range(NBUF):
                g = gg * NBUF + b
                wait(g, b)
                compute(g, b)
                start(g + NBUF, b)
            return 0

        lax.fori_loop(0, (NCHUNK - NBUF) // NBUF, outer, 0)

        for b in range(NBUF):
            g = NCHUNK - NBUF + b
            wait(g, b)
            compute(g, b)

        pltpu.sync_copy(sv, out_hbm.at[pl.ds(wid * PW, PW)])

    return sc_scores


def _loss_sum(scores_2d):
    def body(x_ref, o_ref):
        o_ref[0, 0] = jnp.sum(-jax.nn.log_sigmoid(x_ref[...]))

    out = pl.pallas_call(
        body,
        out_shape=jax.ShapeDtypeStruct((1, 1), jnp.float32),
        out_specs=pl.BlockSpec(memory_space=pltpu.SMEM),
    )(scores_2d)
    return out[0, 0]


def kernel(u_weight, v_weight, pos_u, pos_v, neg_u, neg_v):
    iu = jnp.concatenate([pos_u.reshape(-1), neg_u.reshape(-1)]).astype(jnp.int32)
    iv = jnp.concatenate([pos_v.reshape(-1), neg_v.reshape(-1)]).astype(jnp.int32)
    P = iu.shape[0]
    shp = (NW, P // (NW * CHUNK), CHUNK)
    iug3 = (_PH * (iu // _PB) + iu % _PH).reshape(shp)
    ivg3 = (_PH * (iv // _PB) + iv % _PH).reshape(shp)
    iuo3 = ((iu % _PB) // _PH * EMB_DIM).reshape(shp)
    ivo3 = ((iv % _PB) // _PH * EMB_DIM).reshape(shp)
    u2 = _pack_rows(jnp.swapaxes(u_weight, 0, 1))
    v2 = _pack_rows(jnp.swapaxes(v_weight, 0, 1))
    scores = _make_sc_scores(P)(u2, v2, iug3, ivg3, iuo3, ivo3)
    return _loss_sum(scores.reshape(P // 128, 128))


# speedup vs baseline: 1.1547x; 1.1547x over previous
import functools

import jax
import jax.numpy as jnp
from jax import lax
from jax.experimental import pallas as pl
from jax.experimental.pallas import tpu as pltpu
from jax.experimental.pallas import tpu_sc as plsc

EMB_DIM = 64
NC = 2
NS = 16
NW = NC * NS
CHUNK = 128
NBUF = 2

_PB = 4096
_PH = _PB // 2


def _pack_rows(table_t):
    n = table_t.shape[1]
    grid = (n + _PB - 1) // _PB

    def body(x_ref, o_ref):
        x = x_ref[...]
        eye = (lax.broadcasted_iota(jnp.int32, (EMB_DIM, EMB_DIM), 0)
               == lax.broadcasted_iota(jnp.int32, (EMB_DIM, EMB_DIM), 1)
               ).astype(jnp.float32)
        dims = (((0,), (0,)), ((), ()))
        y = lax.dot_general(x, eye, dims,
                            preferred_element_type=jnp.float32)
        o_ref[...] = jnp.concatenate([y[:_PH], y[_PH:]], axis=1)

    return pl.pallas_call(
        body,
        grid=(grid,),
        in_specs=[pl.BlockSpec((EMB_DIM, _PB), lambda i: (0, i))],
        out_specs=pl.BlockSpec((_PH, 2 * EMB_DIM), lambda i: (i, 0)),
        out_shape=jax.ShapeDtypeStruct((grid * _PH, 2 * EMB_DIM),
                                       jnp.float32),
    )(table_t)


@functools.lru_cache(maxsize=None)
def _make_sc_scores(P: int):
    PW = P // NW
    NCHUNK = PW // CHUNK

    mesh = plsc.VectorSubcoreMesh(
        core_axis_name="c", subcore_axis_name="s",
        num_cores=NC, num_subcores=NS,
    )

    @functools.partial(
        pl.kernel,
        mesh=mesh,
        out_type=jax.ShapeDtypeStruct((P,), jnp.float32),
        scratch_types=[
            pltpu.VMEM((NCHUNK, CHUNK), jnp.int32),
            pltpu.VMEM((NCHUNK, CHUNK), jnp.int32),
            pltpu.VMEM((NCHUNK, CHUNK), jnp.int32),
            pltpu.VMEM((NCHUNK, CHUNK), jnp.int32),
            pltpu.VMEM((NBUF, CHUNK, 2 * EMB_DIM), jnp.float32),
            pltpu.VMEM((NBUF, CHUNK, 2 * EMB_DIM), jnp.float32),
            pltpu.VMEM((PW,), jnp.float32),
            pltpu.SemaphoreType.DMA,
            pltpu.SemaphoreType.DMA,
            pltpu.SemaphoreType.DMA,
            pltpu.SemaphoreType.DMA,
        ],
    )
    def sc_scores(u_hbm, v_hbm, iug_hbm, ivg_hbm, iuo_hbm, ivo_hbm, out_hbm,
                  iug_v, ivg_v, iuo_v, ivo_v, ubuf, vbuf, sv,
                  su0, su1, sv0, sv1):
        sems_u = [su0, su1]
        sems_v = [sv0, sv1]
        wid = lax.axis_index("s") * NC + lax.axis_index("c")

        pltpu.sync_copy(iug_hbm.at[wid], iug_v)
        pltpu.sync_copy(ivg_hbm.at[wid], ivg_v)
        pltpu.sync_copy(iuo_hbm.at[wid], iuo_v)
        pltpu.sync_copy(ivo_hbm.at[wid], ivo_v)

        def start(g, slot):
            pltpu.async_copy(u_hbm.at[iug_v.at[g]], ubuf.at[slot], sems_u[slot])
            pltpu.async_copy(v_hbm.at[ivg_v.at[g]], vbuf.at[slot], sems_v[slot])

        def wait(g, slot):
            pltpu.make_async_copy(
                u_hbm.at[iug_v.at[g]], ubuf.at[slot], sems_u[slot]).wait()
            pltpu.make_async_copy(
                v_hbm.at[ivg_v.at[g]], vbuf.at[slot], sems_v[slot]).wait()

        lanes = lax.iota(jnp.int32, 16)

        def perm(x, idx):
            return lax.gather(
                x, idx[:, None],
                lax.GatherDimensionNumbers(
                    offset_dims=(), collapsed_slice_dims=(0,),
                    start_index_map=(0,)),
                slice_sizes=(1,),
                mode=lax.GatherScatterMode.PROMISE_IN_BOUNDS)

        def compute(g, slot):
            ub = ubuf.at[slot]
            vb = vbuf.at[slot]

            def body(j, _):
                uoff16 = iuo_v[g, pl.ds(j * 16, 16)]
                voff16 = ivo_v[g, pl.ds(j * 16, 16)]
                acc = jnp.zeros((16,), jnp.float32)
                for t in range(16):
                    p = j * 16 + t
                    uo = pl.multiple_of(uoff16[t], EMB_DIM)
                    vo = pl.multiple_of(voff16[t], EMB_DIM)
                    d = jnp.zeros((16,), jnp.float32)
                    for q in range(EMB_DIM // 16):
                        d = d + (ub[p, pl.ds(uo + q * 16, 16)]
                                 * vb[p, pl.ds(vo + q * 16, 16)])
                    for s_ in (8, 4, 2, 1):
                        d = d + perm(d, lanes ^ s_)
                    acc = jnp.where(lanes == t, d, acc)
                sv[pl.ds(g * CHUNK + j * 16, 16)] = acc
                return 0

            lax.fori_loop(0, CHUNK // 16, body, 0)

        for b in range(NBUF):
            start(b, b)

        def outer(gg, _):
            for b in range(NBUF):
                g = gg * NBUF + b
                wait(g, b)
                compute(g, b)
                start(g + NBUF, b)
            return 0

        lax.fori_loop(0, (NCHUNK - NBUF) // NBUF, outer, 0)

        for b in range(NBUF):
            g = NCHUNK - NBUF + b
            wait(g, b)
            compute(g, b)

        pltpu.sync_copy(sv, out_hbm.at[pl.ds(wid * PW, PW)])

    return sc_scores


def _loss_sum(scores_2d):
    def body(x_ref, o_ref):
        o_ref[0, 0] = jnp.sum(-jax.nn.log_sigmoid(x_ref[...]))

    out = pl.pallas_call(
        body,
        out_shape=jax.ShapeDtypeStruct((1, 1), jnp.float32),
        out_specs=pl.BlockSpec(memory_space=pltpu.SMEM),
    )(scores_2d)
    return out[0, 0]


def kernel(u_weight, v_weight, pos_u, pos_v, neg_u, neg_v):
    iu = jnp.concatenate([pos_u.reshape(-1), neg_u.reshape(-1)]).astype(jnp.int32)
    iv = jnp.concatenate([pos_v.reshape(-1), neg_v.reshape(-1)]).astype(jnp.int32)
    P = iu.shape[0]
    shp = (NW, P // (NW * CHUNK), CHUNK)
    iug3 = (_PH * (iu // _PB) + iu % _PH).reshape(shp)
    ivg3 = (_PH * (iv // _PB) + iv % _PH).reshape(shp)
    iuo3 = ((iu % _PB) // _PH * EMB_DIM).reshape(shp)
    ivo3 = ((iv % _PB) // _PH * EMB_DIM).reshape(shp)
    u2 = _pack_rows(jnp.swapaxes(u_weight, 0, 1))
    v2 = _pack_rows(jnp.swapaxes(v_weight, 0, 1))
    scores = _make_sc_scores(P)(u2, v2, iug3, ivg3, iuo3, ivo3)
    return _loss_sum(scores.reshape(P // 128, 128))


# speedup vs baseline: 1.5809x; 1.3691x over previous
import functools

import jax
import jax.numpy as jnp
from jax import lax
from jax.experimental import pallas as pl
from jax.experimental.pallas import tpu as pltpu
from jax.experimental.pallas import tpu_sc as plsc

EMB_DIM = 64
NC = 2
NS = 16
NW = NC * NS
CHUNK = 128
NBUF = 2

_PB = 16384
_PH = _PB // 2


def _pack_rows(table_t):
    n = table_t.shape[1]
    grid = (n + _PB - 1) // _PB

    def body(x_ref, o_ref):
        x = x_ref[...]
        eye = (lax.broadcasted_iota(jnp.int32, (EMB_DIM, EMB_DIM), 0)
               == lax.broadcasted_iota(jnp.int32, (EMB_DIM, EMB_DIM), 1)
               ).astype(jnp.float32)
        dims = (((0,), (0,)), ((), ()))
        y = lax.dot_general(x, eye, dims,
                            preferred_element_type=jnp.float32)
        o_ref[...] = jnp.concatenate([y[:_PH], y[_PH:]], axis=1)

    return pl.pallas_call(
        body,
        grid=(grid,),
        in_specs=[pl.BlockSpec((EMB_DIM, _PB), lambda i: (0, i))],
        out_specs=pl.BlockSpec((_PH, 2 * EMB_DIM), lambda i: (i, 0)),
        out_shape=jax.ShapeDtypeStruct((grid * _PH, 2 * EMB_DIM),
                                       jnp.float32),
    )(table_t)


@functools.lru_cache(maxsize=None)
def _make_sc_scores(P: int):
    PW = P // NW
    NCHUNK = PW // CHUNK

    mesh = plsc.VectorSubcoreMesh(
        core_axis_name="c", subcore_axis_name="s",
        num_cores=NC, num_subcores=NS,
    )

    @functools.partial(
        pl.kernel,
        mesh=mesh,
        out_type=jax.ShapeDtypeStruct((P,), jnp.float32),
        scratch_types=[
            pltpu.VMEM((NCHUNK, CHUNK), jnp.int32),
            pltpu.VMEM((NCHUNK, CHUNK), jnp.int32),
            pltpu.VMEM((NCHUNK, CHUNK), jnp.int32),
            pltpu.VMEM((NCHUNK, CHUNK), jnp.int32),
            pltpu.VMEM((NBUF, CHUNK, 2 * EMB_DIM), jnp.float32),
            pltpu.VMEM((NBUF, CHUNK, 2 * EMB_DIM), jnp.float32),
            pltpu.VMEM((PW,), jnp.float32),
            pltpu.SemaphoreType.DMA,
            pltpu.SemaphoreType.DMA,
            pltpu.SemaphoreType.DMA,
            pltpu.SemaphoreType.DMA,
        ],
    )
    def sc_scores(u_hbm, v_hbm, iug_hbm, ivg_hbm, iuo_hbm, ivo_hbm, out_hbm,
                  iug_v, ivg_v, iuo_v, ivo_v, ubuf, vbuf, sv,
                  su0, su1, sv0, sv1):
        sems_u = [su0, su1]
        sems_v = [sv0, sv1]
        wid = lax.axis_index("s") * NC + lax.axis_index("c")

        pltpu.sync_copy(iug_hbm.at[wid], iug_v)
        pltpu.sync_copy(ivg_hbm.at[wid], ivg_v)
        pltpu.sync_copy(iuo_hbm.at[wid], iuo_v)
        pltpu.sync_copy(ivo_hbm.at[wid], ivo_v)

        def start(g, slot):
            pltpu.async_copy(u_hbm.at[iug_v.at[g]], ubuf.at[slot], sems_u[slot])
            pltpu.async_copy(v_hbm.at[ivg_v.at[g]], vbuf.at[slot], sems_v[slot])

        def wait(g, slot):
            pltpu.make_async_copy(
                u_hbm.at[iug_v.at[g]], ubuf.at[slot], sems_u[slot]).wait()
            pltpu.make_async_copy(
                v_hbm.at[ivg_v.at[g]], vbuf.at[slot], sems_v[slot]).wait()

        lanes = lax.iota(jnp.int32, 16)

        def perm(x, idx):
            return lax.gather(
                x, idx[:, None],
                lax.GatherDimensionNumbers(
                    offset_dims=(), collapsed_slice_dims=(0,),
                    start_index_map=(0,)),
                slice_sizes=(1,),
                mode=lax.GatherScatterMode.PROMISE_IN_BOUNDS)

        def compute(g, slot):
            ub = ubuf.at[slot]
            vb = vbuf.at[slot]

            def body(j, _):
                uoff16 = iuo_v[g, pl.ds(j * 16, 16)]
                voff16 = ivo_v[g, pl.ds(j * 16, 16)]
                acc = jnp.zeros((16,), jnp.float32)
                for t in range(16):
                    p = j * 16 + t
                    uo = pl.multiple_of(uoff16[t], EMB_DIM)
                    vo = pl.multiple_of(voff16[t], EMB_DIM)
                    d = jnp.zeros((16,), jnp.float32)
                    for q in range(EMB_DIM // 16):
                        d = d + (ub[p, pl.ds(uo + q * 16, 16)]
                                 * vb[p, pl.ds(vo + q * 16, 16)])
                    for s_ in (8, 4, 2, 1):
                        d = d + perm(d, lanes ^ s_)
                    acc = jnp.where(lanes == t, d, acc)
                sv[pl.ds(g * CHUNK + j * 16, 16)] = acc
                return 0

            lax.fori_loop(0, CHUNK // 16, body, 0)

        for b in range(NBUF):
            start(b, b)

        def outer(gg, _):
            for b in range(NBUF):
                g = gg * NBUF + b
                wait(g, b)
                compute(g, b)
                start(g + NBUF, b)
            return 0

        lax.fori_loop(0, (NCHUNK - NBUF) // NBUF, outer, 0)

        for b in range(NBUF):
            g = NCHUNK - NBUF + b
            wait(g, b)
            compute(g, b)

        pltpu.sync_copy(sv, out_hbm.at[pl.ds(wid * PW, PW)])

    return sc_scores


def _loss_sum(scores_2d):
    def body(x_ref, o_ref):
        o_ref[0, 0] = jnp.sum(-jax.nn.log_sigmoid(x_ref[...]))

    out = pl.pallas_call(
        body,
        out_shape=jax.ShapeDtypeStruct((1, 1), jnp.float32),
        out_specs=pl.BlockSpec(memory_space=pltpu.SMEM),
    )(scores_2d)
    return out[0, 0]


def kernel(u_weight, v_weight, pos_u, pos_v, neg_u, neg_v):
    iu = jnp.concatenate([pos_u.reshape(-1), neg_u.reshape(-1)]).astype(jnp.int32)
    iv = jnp.concatenate([pos_v.reshape(-1), neg_v.reshape(-1)]).astype(jnp.int32)
    P = iu.shape[0]
    shp = (NW, P // (NW * CHUNK), CHUNK)
    iug3 = (_PH * (iu // _PB) + iu % _PH).reshape(shp)
    ivg3 = (_PH * (iv // _PB) + iv % _PH).reshape(shp)
    iuo3 = ((iu % _PB) // _PH * EMB_DIM).reshape(shp)
    ivo3 = ((iv % _PB) // _PH * EMB_DIM).reshape(shp)
    u2 = _pack_rows(jnp.swapaxes(u_weight, 0, 1))
    v2 = _pack_rows(jnp.swapaxes(v_weight, 0, 1))
    scores = _make_sc_scores(P)(u2, v2, iug3, ivg3, iuo3, ivo3)
    return _loss_sum(scores.reshape(P // 128, 128))


# speedup vs baseline: 1.6689x; 1.0557x over previous
import functools

import jax
import jax.numpy as jnp
from jax import lax
from jax.experimental import pallas as pl
from jax.experimental.pallas import tpu as pltpu
from jax.experimental.pallas import tpu_sc as plsc

EMB_DIM = 64
NC = 2
NS = 16
NW = NC * NS
CHUNK = 128
NBUF = 2

_PB = 32768
_PH = _PB // 2


def _pack_rows(table_t):
    n = table_t.shape[1]
    grid = (n + _PB - 1) // _PB

    def body(x_ref, o_ref):
        x = x_ref[...]
        eye = (lax.broadcasted_iota(jnp.int32, (EMB_DIM, EMB_DIM), 0)
               == lax.broadcasted_iota(jnp.int32, (EMB_DIM, EMB_DIM), 1)
               ).astype(jnp.float32)
        dims = (((0,), (0,)), ((), ()))
        y = lax.dot_general(x, eye, dims,
                            preferred_element_type=jnp.float32)
        o_ref[...] = jnp.concatenate([y[:_PH], y[_PH:]], axis=1)

    return pl.pallas_call(
        body,
        grid=(grid,),
        in_specs=[pl.BlockSpec((EMB_DIM, _PB), lambda i: (0, i))],
        out_specs=pl.BlockSpec((_PH, 2 * EMB_DIM), lambda i: (i, 0)),
        out_shape=jax.ShapeDtypeStruct((grid * _PH, 2 * EMB_DIM),
                                       jnp.float32),
    )(table_t)


@functools.lru_cache(maxsize=None)
def _make_sc_scores(P: int):
    PW = P // NW
    NCHUNK = PW // CHUNK

    mesh = plsc.VectorSubcoreMesh(
        core_axis_name="c", subcore_axis_name="s",
        num_cores=NC, num_subcores=NS,
    )

    @functools.partial(
        pl.kernel,
        mesh=mesh,
        out_type=jax.ShapeDtypeStruct((P,), jnp.float32),
        scratch_types=[
            pltpu.VMEM((NCHUNK, CHUNK), jnp.int32),
            pltpu.VMEM((NCHUNK, CHUNK), jnp.int32),
            pltpu.VMEM((NCHUNK, CHUNK), jnp.int32),
            pltpu.VMEM((NCHUNK, CHUNK), jnp.int32),
            pltpu.VMEM((NBUF, CHUNK, 2 * EMB_DIM), jnp.float32),
            pltpu.VMEM((NBUF, CHUNK, 2 * EMB_DIM), jnp.float32),
            pltpu.VMEM((PW,), jnp.float32),
            pltpu.SemaphoreType.DMA,
            pltpu.SemaphoreType.DMA,
            pltpu.SemaphoreType.DMA,
            pltpu.SemaphoreType.DMA,
        ],
    )
    def sc_scores(u_hbm, v_hbm, iug_hbm, ivg_hbm, iuo_hbm, ivo_hbm, out_hbm,
                  iug_v, ivg_v, iuo_v, ivo_v, ubuf, vbuf, sv,
                  su0, su1, sv0, sv1):
        sems_u = [su0, su1]
        sems_v = [sv0, sv1]
        wid = lax.axis_index("s") * NC + lax.axis_index("c")

        pltpu.sync_copy(iug_hbm.at[wid], iug_v)
        pltpu.sync_copy(ivg_hbm.at[wid], ivg_v)
        pltpu.sync_copy(iuo_hbm.at[wid], iuo_v)
        pltpu.sync_copy(ivo_hbm.at[wid], ivo_v)

        def start(g, slot):
            pltpu.async_copy(u_hbm.at[iug_v.at[g]], ubuf.at[slot], sems_u[slot])
            pltpu.async_copy(v_hbm.at[ivg_v.at[g]], vbuf.at[slot], sems_v[slot])

        def wait(g, slot):
            pltpu.make_async_copy(
                u_hbm.at[iug_v.at[g]], ubuf.at[slot], sems_u[slot]).wait()
            pltpu.make_async_copy(
                v_hbm.at[ivg_v.at[g]], vbuf.at[slot], sems_v[slot]).wait()

        lanes = lax.iota(jnp.int32, 16)

        def perm(x, idx):
            return lax.gather(
                x, idx[:, None],
                lax.GatherDimensionNumbers(
                    offset_dims=(), collapsed_slice_dims=(0,),
                    start_index_map=(0,)),
                slice_sizes=(1,),
                mode=lax.GatherScatterMode.PROMISE_IN_BOUNDS)

        def compute(g, slot):
            ub = ubuf.at[slot]
            vb = vbuf.at[slot]

            def body(j, _):
                uoff16 = iuo_v[g, pl.ds(j * 16, 16)]
                voff16 = ivo_v[g, pl.ds(j * 16, 16)]
                acc = jnp.zeros((16,), jnp.float32)
                for t in range(16):
                    p = j * 16 + t
                    uo = pl.multiple_of(uoff16[t], EMB_DIM)
                    vo = pl.multiple_of(voff16[t], EMB_DIM)
                    d = jnp.zeros((16,), jnp.float32)
                    for q in range(EMB_DIM // 16):
                        d = d + (ub[p, pl.ds(uo + q * 16, 16)]
                                 * vb[p, pl.ds(vo + q * 16, 16)])
                    for s_ in (8, 4, 2, 1):
                        d = d + perm(d, lanes ^ s_)
                    acc = jnp.where(lanes == t, d, acc)
                sv[pl.ds(g * CHUNK + j * 16, 16)] = acc
                return 0

            lax.fori_loop(0, CHUNK // 16, body, 0)

        for b in range(NBUF):
            start(b, b)

        def outer(gg, _):
            for b in range(NBUF):
                g = gg * NBUF + b
                wait(g, b)
                compute(g, b)
                start(g + NBUF, b)
            return 0

        lax.fori_loop(0, (NCHUNK - NBUF) // NBUF, outer, 0)

        for b in range(NBUF):
            g = NCHUNK - NBUF + b
            wait(g, b)
            compute(g, b)

        pltpu.sync_copy(sv, out_hbm.at[pl.ds(wid * PW, PW)])

    return sc_scores


def _loss_sum(scores_2d):
    def body(x_ref, o_ref):
        o_ref[0, 0] = jnp.sum(-jax.nn.log_sigmoid(x_ref[...]))

    out = pl.pallas_call(
        body,
        out_shape=jax.ShapeDtypeStruct((1, 1), jnp.float32),
        out_specs=pl.BlockSpec(memory_space=pltpu.SMEM),
    )(scores_2d)
    return out[0, 0]


def kernel(u_weight, v_weight, pos_u, pos_v, neg_u, neg_v):
    iu = jnp.concatenate([pos_u.reshape(-1), neg_u.reshape(-1)]).astype(jnp.int32)
    iv = jnp.concatenate([pos_v.reshape(-1), neg_v.reshape(-1)]).astype(jnp.int32)
    P = iu.shape[0]
    shp = (NW, P // (NW * CHUNK), CHUNK)
    iug3 = (_PH * (iu // _PB) + iu % _PH).reshape(shp)
    ivg3 = (_PH * (iv // _PB) + iv % _PH).reshape(shp)
    iuo3 = ((iu % _PB) // _PH * EMB_DIM).reshape(shp)
    ivo3 = ((iv % _PB) // _PH * EMB_DIM).reshape(shp)
    u2 = _pack_rows(jnp.swapaxes(u_weight, 0, 1))
    v2 = _pack_rows(jnp.swapaxes(v_weight, 0, 1))
    scores = _make_sc_scores(P)(u2, v2, iug3, ivg3, iuo3, ivo3)
    return _loss_sum(scores.reshape(P // 128, 128))


# speedup vs baseline: 1.6704x; 1.0009x over previous
import functools

import jax
import jax.numpy as jnp
from jax import lax
from jax.experimental import pallas as pl
from jax.experimental.pallas import tpu as pltpu
from jax.experimental.pallas import tpu_sc as plsc

EMB_DIM = 64
NC = 2
NS = 16
NW = NC * NS
CHUNK = 128
NBUF = 2

_PB = 32768
_PH = _PB // 2


def _pack_rows(table_t):
    n = table_t.shape[1]
    grid = (n + _PB - 1) // _PB

    def body(x_ref, o_ref):
        eye = (lax.broadcasted_iota(jnp.int32, (EMB_DIM, EMB_DIM), 0)
               == lax.broadcasted_iota(jnp.int32, (EMB_DIM, EMB_DIM), 1)
               ).astype(jnp.float32)
        dims = (((0,), (0,)), ((), ()))
        o_ref[:, :EMB_DIM] = lax.dot_general(
            x_ref[:, :_PH], eye, dims, preferred_element_type=jnp.float32)
        o_ref[:, EMB_DIM:] = lax.dot_general(
            x_ref[:, _PH:], eye, dims, preferred_element_type=jnp.float32)

    return pl.pallas_call(
        body,
        grid=(grid,),
        in_specs=[pl.BlockSpec((EMB_DIM, _PB), lambda i: (0, i))],
        out_specs=pl.BlockSpec((_PH, 2 * EMB_DIM), lambda i: (i, 0)),
        out_shape=jax.ShapeDtypeStruct((grid * _PH, 2 * EMB_DIM),
                                       jnp.float32),
    )(table_t)


@functools.lru_cache(maxsize=None)
def _make_sc_scores(P: int):
    PW = P // NW
    NCHUNK = PW // CHUNK

    mesh = plsc.VectorSubcoreMesh(
        core_axis_name="c", subcore_axis_name="s",
        num_cores=NC, num_subcores=NS,
    )

    @functools.partial(
        pl.kernel,
        mesh=mesh,
        out_type=jax.ShapeDtypeStruct((P,), jnp.float32),
        scratch_types=[
            pltpu.VMEM((NCHUNK, CHUNK), jnp.int32),
            pltpu.VMEM((NCHUNK, CHUNK), jnp.int32),
            pltpu.VMEM((NCHUNK, CHUNK), jnp.int32),
            pltpu.VMEM((NCHUNK, CHUNK), jnp.int32),
            pltpu.VMEM((NBUF, CHUNK, 2 * EMB_DIM), jnp.float32),
            pltpu.VMEM((NBUF, CHUNK, 2 * EMB_DIM), jnp.float32),
            pltpu.VMEM((PW,), jnp.float32),
            pltpu.SemaphoreType.DMA,
            pltpu.SemaphoreType.DMA,
            pltpu.SemaphoreType.DMA,
            pltpu.SemaphoreType.DMA,
        ],
    )
    def sc_scores(u_hbm, v_hbm, iug_hbm, ivg_hbm, iuo_hbm, ivo_hbm, out_hbm,
                  iug_v, ivg_v, iuo_v, ivo_v, ubuf, vbuf, sv,
                  su0, su1, sv0, sv1):
        sems_u = [su0, su1]
        sems_v = [sv0, sv1]
        wid = lax.axis_index("s") * NC + lax.axis_index("c")

        pltpu.sync_copy(iug_hbm.at[wid], iug_v)
        pltpu.sync_copy(ivg_hbm.at[wid], ivg_v)
        pltpu.sync_copy(iuo_hbm.at[wid], iuo_v)
        pltpu.sync_copy(ivo_hbm.at[wid], ivo_v)

        def start(g, slot):
            pltpu.async_copy(u_hbm.at[iug_v.at[g]], ubuf.at[slot], sems_u[slot])
            pltpu.async_copy(v_hbm.at[ivg_v.at[g]], vbuf.at[slot], sems_v[slot])

        def wait(g, slot):
            pltpu.make_async_copy(
                u_hbm.at[iug_v.at[g]], ubuf.at[slot], sems_u[slot]).wait()
            pltpu.make_async_copy(
                v_hbm.at[ivg_v.at[g]], vbuf.at[slot], sems_v[slot]).wait()

        lanes = lax.iota(jnp.int32, 16)

        def perm(x, idx):
            return lax.gather(
                x, idx[:, None],
                lax.GatherDimensionNumbers(
                    offset_dims=(), collapsed_slice_dims=(0,),
                    start_index_map=(0,)),
                slice_sizes=(1,),
                mode=lax.GatherScatterMode.PROMISE_IN_BOUNDS)

        def compute(g, slot):
            ub = ubuf.at[slot]
            vb = vbuf.at[slot]

            def body(j, _):
                uoff16 = iuo_v[g, pl.ds(j * 16, 16)]
                voff16 = ivo_v[g, pl.ds(j * 16, 16)]
                acc = jnp.zeros((16,), jnp.float32)
                for t in range(16):
                    p = j * 16 + t
                    uo = pl.multiple_of(uoff16[t], EMB_DIM)
                    vo = pl.multiple_of(voff16[t], EMB_DIM)
                    d = jnp.zeros((16,), jnp.float32)
                    for q in range(EMB_DIM // 16):
                        d = d + (ub[p, pl.ds(uo + q * 16, 16)]
                                 * vb[p, pl.ds(vo + q * 16, 16)])
                    for s_ in (8, 4, 2, 1):
                        d = d + perm(d, lanes ^ s_)
                    acc = jnp.where(lanes == t, d, acc)
                sv[pl.ds(g * CHUNK + j * 16, 16)] = acc
                return 0

            lax.fori_loop(0, CHUNK // 16, body, 0)

        for b in range(NBUF):
            start(b, b)

        def outer(gg, _):
            for b in range(NBUF):
                g = gg * NBUF + b
                wait(g, b)
                compute(g, b)
                start(g + NBUF, b)
            return 0

        lax.fori_loop(0, (NCHUNK - NBUF) // NBUF, outer, 0)

        for b in range(NBUF):
            g = NCHUNK - NBUF + b
            wait(g, b)
            compute(g, b)

        pltpu.sync_copy(sv, out_hbm.at[pl.ds(wid * PW, PW)])

    return sc_scores


def _loss_sum(scores_2d):
    def body(x_ref, o_ref):
        o_ref[0, 0] = jnp.sum(-jax.nn.log_sigmoid(x_ref[...]))

    out = pl.pallas_call(
        body,
        out_shape=jax.ShapeDtypeStruct((1, 1), jnp.float32),
        out_specs=pl.BlockSpec(memory_space=pltpu.SMEM),
    )(scores_2d)
    return out[0, 0]


def kernel(u_weight, v_weight, pos_u, pos_v, neg_u, neg_v):
    iu = jnp.concatenate([pos_u.reshape(-1), neg_u.reshape(-1)]).astype(jnp.int32)
    iv = jnp.concatenate([pos_v.reshape(-1), neg_v.reshape(-1)]).astype(jnp.int32)
    P = iu.shape[0]
    shp = (NW, P // (NW * CHUNK), CHUNK)
    iug3 = (_PH * (iu // _PB) + iu % _PH).reshape(shp)
    ivg3 = (_PH * (iv // _PB) + iv % _PH).reshape(shp)
    iuo3 = ((iu % _PB) // _PH * EMB_DIM).reshape(shp)
    ivo3 = ((iv % _PB) // _PH * EMB_DIM).reshape(shp)
    u2 = _pack_rows(jnp.swapaxes(u_weight, 0, 1))
    v2 = _pack_rows(jnp.swapaxes(v_weight, 0, 1))
    scores = _make_sc_scores(P)(u2, v2, iug3, ivg3, iuo3, ivo3)
    return _loss_sum(scores.reshape(P // 128, 128))


# speedup vs baseline: 1.7469x; 1.0458x over previous
import functools

import jax
import jax.numpy as jnp
from jax import lax
from jax.experimental import pallas as pl
from jax.experimental.pallas import tpu as pltpu
from jax.experimental.pallas import tpu_sc as plsc

EMB_DIM = 64
NC = 2
NS = 16
NW = NC * NS
CHUNK = 128
NBUF = 2

_PB = 32768
_PH = _PB // 2


def _pack_rows(table_t):
    n = table_t.shape[1]
    grid = (n + _PB - 1) // _PB

    def body(x_ref, o_ref):
        eye = (lax.broadcasted_iota(jnp.int32, (EMB_DIM, EMB_DIM), 0)
               == lax.broadcasted_iota(jnp.int32, (EMB_DIM, EMB_DIM), 1)
               ).astype(jnp.float32)
        dims = (((0,), (0,)), ((), ()))
        o_ref[:, :EMB_DIM] = lax.dot_general(
            x_ref[:, :_PH], eye, dims, preferred_element_type=jnp.float32)
        o_ref[:, EMB_DIM:] = lax.dot_general(
            x_ref[:, _PH:], eye, dims, preferred_element_type=jnp.float32)

    return pl.pallas_call(
        body,
        grid=(grid,),
        in_specs=[pl.BlockSpec((EMB_DIM, _PB), lambda i: (0, i))],
        out_specs=pl.BlockSpec((_PH, 2 * EMB_DIM), lambda i: (i, 0)),
        out_shape=jax.ShapeDtypeStruct((grid * _PH, 2 * EMB_DIM),
                                       jnp.float32),
    )(table_t)


@functools.lru_cache(maxsize=None)
def _make_sc_scores(P: int):
    PW = P // NW
    NCHUNK = PW // CHUNK

    mesh = plsc.VectorSubcoreMesh(
        core_axis_name="c", subcore_axis_name="s",
        num_cores=NC, num_subcores=NS,
    )

    @functools.partial(
        pl.kernel,
        mesh=mesh,
        out_type=jax.ShapeDtypeStruct((P,), jnp.float32),
        scratch_types=[
            pltpu.VMEM((NCHUNK, CHUNK), jnp.int32),
            pltpu.VMEM((NCHUNK, CHUNK), jnp.int32),
            pltpu.VMEM((NCHUNK, CHUNK), jnp.int32),
            pltpu.VMEM((NCHUNK, CHUNK), jnp.int32),
            pltpu.VMEM((NBUF, CHUNK, 2 * EMB_DIM), jnp.float32),
            pltpu.VMEM((NBUF, CHUNK, 2 * EMB_DIM), jnp.float32),
            pltpu.VMEM((PW,), jnp.float32),
            pltpu.SemaphoreType.DMA,
            pltpu.SemaphoreType.DMA,
            pltpu.SemaphoreType.DMA,
            pltpu.SemaphoreType.DMA,
        ],
    )
    def sc_scores(u_hbm, v_hbm, iug_hbm, ivg_hbm, iuo_hbm, ivo_hbm, out_hbm,
                  iug_v, ivg_v, iuo_v, ivo_v, ubuf, vbuf, sv,
                  su0, su1, sv0, sv1):
        sems_u = [su0, su1]
        sems_v = [sv0, sv1]
        wid = lax.axis_index("s") * NC + lax.axis_index("c")

        pltpu.sync_copy(iug_hbm.at[wid], iug_v)
        pltpu.sync_copy(ivg_hbm.at[wid], ivg_v)
        pltpu.sync_copy(iuo_hbm.at[wid], iuo_v)
        pltpu.sync_copy(ivo_hbm.at[wid], ivo_v)

        def start(g, slot):
            pltpu.async_copy(u_hbm.at[iug_v.at[g]], ubuf.at[slot], sems_u[slot])
            pltpu.async_copy(v_hbm.at[ivg_v.at[g]], vbuf.at[slot], sems_v[slot])

        def wait(g, slot):
            pltpu.make_async_copy(
                u_hbm.at[iug_v.at[g]], ubuf.at[slot], sems_u[slot]).wait()
            pltpu.make_async_copy(
                v_hbm.at[ivg_v.at[g]], vbuf.at[slot], sems_v[slot]).wait()

        lanes = lax.iota(jnp.int32, 16)

        def perm(x, idx):
            return lax.gather(
                x, idx[:, None],
                lax.GatherDimensionNumbers(
                    offset_dims=(), collapsed_slice_dims=(0,),
                    start_index_map=(0,)),
                slice_sizes=(1,),
                mode=lax.GatherScatterMode.PROMISE_IN_BOUNDS)

        def compute(g, slot):
            ub = ubuf.at[slot]
            vb = vbuf.at[slot]

            def body(j, _):
                uoff16 = iuo_v[g, pl.ds(j * 16, 16)]
                voff16 = ivo_v[g, pl.ds(j * 16, 16)]
                acc = jnp.zeros((16,), jnp.float32)
                for t in range(16):
                    p = j * 16 + t
                    uo = pl.multiple_of(uoff16[t], EMB_DIM)
                    vo = pl.multiple_of(voff16[t], EMB_DIM)
                    d = jnp.zeros((16,), jnp.float32)
                    for q in range(EMB_DIM // 16):
                        d = d + (ub[p, pl.ds(uo + q * 16, 16)]
                                 * vb[p, pl.ds(vo + q * 16, 16)])
                    for s_ in (8, 4, 2, 1):
                        d = d + perm(d, lanes ^ s_)
                    acc = jnp.where(lanes == t, d, acc)
                sv[pl.ds(g * CHUNK + j * 16, 16)] = acc
                return 0

            lax.fori_loop(0, CHUNK // 16, body, 0)

        for b in range(NBUF):
            start(b, b)

        def outer(gg, _):
            for b in range(NBUF):
                g = gg * NBUF + b
                wait(g, b)
                compute(g, b)
                start(g + NBUF, b)
            return 0

        lax.fori_loop(0, (NCHUNK - NBUF) // NBUF, outer, 0)

        for b in range(NBUF):
            g = NCHUNK - NBUF + b
            wait(g, b)
            compute(g, b)

        pltpu.sync_copy(sv, out_hbm.at[pl.ds(wid * PW, PW)])

    return sc_scores


def _loss_sum(scores_2d):
    def body(x_ref, o_ref):
        o_ref[0, 0] = jnp.sum(-jax.nn.log_sigmoid(x_ref[...]))

    out = pl.pallas_call(
        body,
        out_shape=jax.ShapeDtypeStruct((1, 1), jnp.float32),
        out_specs=pl.BlockSpec(memory_space=pltpu.SMEM),
    )(scores_2d)
    return out[0, 0]


def kernel(u_weight, v_weight, pos_u, pos_v, neg_u, neg_v):
    iu = jnp.concatenate(
        [pos_u.reshape(1, -1), jnp.swapaxes(neg_u, 0, 1)]).astype(jnp.int32)
    iv = jnp.concatenate(
        [pos_v.reshape(1, -1), jnp.swapaxes(neg_v, 0, 1)]).astype(jnp.int32)
    P = iu.size
    shp = (NW, P // (NW * CHUNK), CHUNK)
    iug3 = (_PH * (iu // _PB) + iu % _PH).reshape(shp)
    ivg3 = (_PH * (iv // _PB) + iv % _PH).reshape(shp)
    iuo3 = ((iu % _PB) // _PH * EMB_DIM).reshape(shp)
    ivo3 = ((iv % _PB) // _PH * EMB_DIM).reshape(shp)
    u2 = _pack_rows(jnp.swapaxes(u_weight, 0, 1))
    v2 = _pack_rows(jnp.swapaxes(v_weight, 0, 1))
    scores = _make_sc_scores(P)(u2, v2, iug3, ivg3, iuo3, ivo3)
    return _loss_sum(scores.reshape(P // 128, 128))


# speedup vs baseline: 1.7730x; 1.0149x over previous
import functools

import jax
import jax.numpy as jnp
from jax import lax
from jax.experimental import pallas as pl
from jax.experimental.pallas import tpu as pltpu
from jax.experimental.pallas import tpu_sc as plsc

EMB_DIM = 64
NC = 2
NS = 16
NW = NC * NS
CHUNK = 128
NBUF = 3

_PB = 32768
_PH = _PB // 2


def _pack_rows(table_t):
    n = table_t.shape[1]
    grid = (n + _PB - 1) // _PB

    def body(x_ref, o_ref):
        eye = (lax.broadcasted_iota(jnp.int32, (EMB_DIM, EMB_DIM), 0)
               == lax.broadcasted_iota(jnp.int32, (EMB_DIM, EMB_DIM), 1)
               ).astype(jnp.float32)
        dims = (((0,), (0,)), ((), ()))
        o_ref[:, :EMB_DIM] = lax.dot_general(
            x_ref[:, :_PH], eye, dims, preferred_element_type=jnp.float32)
        o_ref[:, EMB_DIM:] = lax.dot_general(
            x_ref[:, _PH:], eye, dims, preferred_element_type=jnp.float32)

    return pl.pallas_call(
        body,
        grid=(grid,),
        in_specs=[pl.BlockSpec((EMB_DIM, _PB), lambda i: (0, i))],
        out_specs=pl.BlockSpec((_PH, 2 * EMB_DIM), lambda i: (i, 0)),
        out_shape=jax.ShapeDtypeStruct((grid * _PH, 2 * EMB_DIM),
                                       jnp.float32),
    )(table_t)


@functools.lru_cache(maxsize=None)
def _make_sc_scores(P: int):
    PW = P // NW
    NCHUNK = PW // CHUNK

    mesh = plsc.VectorSubcoreMesh(
        core_axis_name="c", subcore_axis_name="s",
        num_cores=NC, num_subcores=NS,
    )

    @functools.partial(
        pl.kernel,
        mesh=mesh,
        out_type=jax.ShapeDtypeStruct((P,), jnp.float32),
        scratch_types=[
            pltpu.VMEM((NCHUNK, CHUNK), jnp.int32),
            pltpu.VMEM((NCHUNK, CHUNK), jnp.int32),
            pltpu.VMEM((NCHUNK, CHUNK), jnp.int32),
            pltpu.VMEM((NCHUNK, CHUNK), jnp.int32),
            pltpu.VMEM((NBUF, CHUNK, 2 * EMB_DIM), jnp.float32),
            pltpu.VMEM((NBUF, CHUNK, 2 * EMB_DIM), jnp.float32),
            pltpu.VMEM((PW,), jnp.float32),
        ] + [pltpu.SemaphoreType.DMA] * (2 * NBUF),
    )
    def sc_scores(u_hbm, v_hbm, iug_hbm, ivg_hbm, iuo_hbm, ivo_hbm, out_hbm,
                  iug_v, ivg_v, iuo_v, ivo_v, ubuf, vbuf, sv, *sems):
        sems_u = list(sems[:NBUF])
        sems_v = list(sems[NBUF:])
        wid = lax.axis_index("s") * NC + lax.axis_index("c")

        pltpu.sync_copy(iug_hbm.at[wid], iug_v)
        pltpu.sync_copy(ivg_hbm.at[wid], ivg_v)
        pltpu.sync_copy(iuo_hbm.at[wid], iuo_v)
        pltpu.sync_copy(ivo_hbm.at[wid], ivo_v)

        def start(g, slot):
            pltpu.async_copy(u_hbm.at[iug_v.at[g]], ubuf.at[slot], sems_u[slot])
            pltpu.async_copy(v_hbm.at[ivg_v.at[g]], vbuf.at[slot], sems_v[slot])

        def wait(g, slot):
            pltpu.make_async_copy(
                u_hbm.at[iug_v.at[g]], ubuf.at[slot], sems_u[slot]).wait()
            pltpu.make_async_copy(
                v_hbm.at[ivg_v.at[g]], vbuf.at[slot], sems_v[slot]).wait()

        lanes = lax.iota(jnp.int32, 16)

        def perm(x, idx):
            return lax.gather(
                x, idx[:, None],
                lax.GatherDimensionNumbers(
                    offset_dims=(), collapsed_slice_dims=(0,),
                    start_index_map=(0,)),
                slice_sizes=(1,),
                mode=lax.GatherScatterMode.PROMISE_IN_BOUNDS)

        def compute(g, slot):
            ub = ubuf.at[slot]
            vb = vbuf.at[slot]

            def body(j, _):
                uoff16 = iuo_v[g, pl.ds(j * 16, 16)]
                voff16 = ivo_v[g, pl.ds(j * 16, 16)]
                acc = jnp.zeros((16,), jnp.float32)
                for t in range(16):
                    p = j * 16 + t
                    uo = pl.multiple_of(uoff16[t], EMB_DIM)
                    vo = pl.multiple_of(voff16[t], EMB_DIM)
                    d = jnp.zeros((16,), jnp.float32)
                    for q in range(EMB_DIM // 16):
                        d = d + (ub[p, pl.ds(uo + q * 16, 16)]
                                 * vb[p, pl.ds(vo + q * 16, 16)])
                    for s_ in (8, 4, 2, 1):
                        d = d + perm(d, lanes ^ s_)
                    acc = jnp.where(lanes == t, d, acc)
                sv[pl.ds(g * CHUNK + j * 16, 16)] = acc
                return 0

            lax.fori_loop(0, CHUNK // 16, body, 0)

        for b in range(NBUF):
            start(b, b)

        def outer(gg, _):
            for b in range(NBUF):
                g = gg * NBUF + b
                wait(g, b)
                compute(g, b)
                start(g + NBUF, b)
            return 0

        lax.fori_loop(0, (NCHUNK - NBUF) // NBUF, outer, 0)

        for b in range(NBUF):
            g = NCHUNK - NBUF + b
            wait(g, b)
            compute(g, b)

        pltpu.sync_copy(sv, out_hbm.at[pl.ds(wid * PW, PW)])

    return sc_scores


def _loss_sum(scores_2d):
    def body(x_ref, o_ref):
        o_ref[0, 0] = jnp.sum(-jax.nn.log_sigmoid(x_ref[...]))

    out = pl.pallas_call(
        body,
        out_shape=jax.ShapeDtypeStruct((1, 1), jnp.float32),
        out_specs=pl.BlockSpec(memory_space=pltpu.SMEM),
    )(scores_2d)
    return out[0, 0]


def kernel(u_weight, v_weight, pos_u, pos_v, neg_u, neg_v):
    iu = jnp.concatenate(
        [pos_u.reshape(1, -1), jnp.swapaxes(neg_u, 0, 1)]).astype(jnp.int32)
    iv = jnp.concatenate(
        [pos_v.reshape(1, -1), jnp.swapaxes(neg_v, 0, 1)]).astype(jnp.int32)
    P = iu.size
    shp = (NW, P // (NW * CHUNK), CHUNK)
    iug3 = (_PH * (iu // _PB) + iu % _PH).reshape(shp)
    ivg3 = (_PH * (iv // _PB) + iv % _PH).reshape(shp)
    iuo3 = ((iu % _PB) // _PH * EMB_DIM).reshape(shp)
    ivo3 = ((iv % _PB) // _PH * EMB_DIM).reshape(shp)
    u2 = _pack_rows(jnp.swapaxes(u_weight, 0, 1))
    v2 = _pack_rows(jnp.swapaxes(v_weight, 0, 1))
    scores = _make_sc_scores(P)(u2, v2, iug3, ivg3, iuo3, ivo3)
    return _loss_sum(scores.reshape(P // 128, 128))


# speedup vs baseline: 1.7829x; 1.0056x over previous
import functools

import jax
import jax.numpy as jnp
from jax import lax
from jax.experimental import pallas as pl
from jax.experimental.pallas import tpu as pltpu
from jax.experimental.pallas import tpu_sc as plsc

EMB_DIM = 64
NC = 2
NS = 16
NW = NC * NS
CHUNK = 128
NBUF = 3

_PB = 32768
_PH = _PB // 2


def _pack_rows(table_t):
    n = table_t.shape[1]
    grid = (n + _PB - 1) // _PB

    def body(x_ref, o_ref):
        o_ref[:, :EMB_DIM] = jnp.swapaxes(x_ref[:, :_PH], 0, 1)
        o_ref[:, EMB_DIM:] = jnp.swapaxes(x_ref[:, _PH:], 0, 1)

    return pl.pallas_call(
        body,
        grid=(grid,),
        in_specs=[pl.BlockSpec((EMB_DIM, _PB), lambda i: (0, i))],
        out_specs=pl.BlockSpec((_PH, 2 * EMB_DIM), lambda i: (i, 0)),
        out_shape=jax.ShapeDtypeStruct((grid * _PH, 2 * EMB_DIM),
                                       jnp.float32),
    )(table_t)


@functools.lru_cache(maxsize=None)
def _make_sc_scores(P: int):
    PW = P // NW
    NCHUNK = PW // CHUNK

    mesh = plsc.VectorSubcoreMesh(
        core_axis_name="c", subcore_axis_name="s",
        num_cores=NC, num_subcores=NS,
    )

    @functools.partial(
        pl.kernel,
        mesh=mesh,
        out_type=jax.ShapeDtypeStruct((P,), jnp.float32),
        scratch_types=[
            pltpu.VMEM((NCHUNK, CHUNK), jnp.int32),
            pltpu.VMEM((NCHUNK, CHUNK), jnp.int32),
            pltpu.VMEM((NCHUNK, CHUNK), jnp.int32),
            pltpu.VMEM((NCHUNK, CHUNK), jnp.int32),
            pltpu.VMEM((NBUF, CHUNK, 2 * EMB_DIM), jnp.float32),
            pltpu.VMEM((NBUF, CHUNK, 2 * EMB_DIM), jnp.float32),
            pltpu.VMEM((PW,), jnp.float32),
        ] + [pltpu.SemaphoreType.DMA] * (2 * NBUF),
    )
    def sc_scores(u_hbm, v_hbm, iug_hbm, ivg_hbm, iuo_hbm, ivo_hbm, out_hbm,
                  iug_v, ivg_v, iuo_v, ivo_v, ubuf, vbuf, sv, *sems):
        sems_u = list(sems[:NBUF])
        sems_v = list(sems[NBUF:])
        wid = lax.axis_index("s") * NC + lax.axis_index("c")

        pltpu.sync_copy(iug_hbm.at[wid], iug_v)
        pltpu.sync_copy(ivg_hbm.at[wid], ivg_v)
        pltpu.sync_copy(iuo_hbm.at[wid], iuo_v)
        pltpu.sync_copy(ivo_hbm.at[wid], ivo_v)

        def start(g, slot):
            pltpu.async_copy(u_hbm.at[iug_v.at[g]], ubuf.at[slot], sems_u[slot])
            pltpu.async_copy(v_hbm.at[ivg_v.at[g]], vbuf.at[slot], sems_v[slot])

        def wait(g, slot):
            pltpu.make_async_copy(
                u_hbm.at[iug_v.at[g]], ubuf.at[slot], sems_u[slot]).wait()
            pltpu.make_async_copy(
                v_hbm.at[ivg_v.at[g]], vbuf.at[slot], sems_v[slot]).wait()

        lanes = lax.iota(jnp.int32, 16)

        def perm(x, idx):
            return lax.gather(
                x, idx[:, None],
                lax.GatherDimensionNumbers(
                    offset_dims=(), collapsed_slice_dims=(0,),
                    start_index_map=(0,)),
                slice_sizes=(1,),
                mode=lax.GatherScatterMode.PROMISE_IN_BOUNDS)

        def compute(g, slot):
            ub = ubuf.at[slot]
            vb = vbuf.at[slot]

            def body(j, _):
                uoff16 = iuo_v[g, pl.ds(j * 16, 16)]
                voff16 = ivo_v[g, pl.ds(j * 16, 16)]
                acc = jnp.zeros((16,), jnp.float32)
                for t in range(16):
                    p = j * 16 + t
                    uo = pl.multiple_of(uoff16[t], EMB_DIM)
                    vo = pl.multiple_of(voff16[t], EMB_DIM)
                    d = jnp.zeros((16,), jnp.float32)
                    for q in range(EMB_DIM // 16):
                        d = d + (ub[p, pl.ds(uo + q * 16, 16)]
                                 * vb[p, pl.ds(vo + q * 16, 16)])
                    for s_ in (8, 4, 2, 1):
                        d = d + perm(d, lanes ^ s_)
                    acc = jnp.where(lanes == t, d, acc)
                sv[pl.ds(g * CHUNK + j * 16, 16)] = acc
                return 0

            lax.fori_loop(0, CHUNK // 16, body, 0)

        for b in range(NBUF):
            start(b, b)

        def outer(gg, _):
            for b in range(NBUF):
                g = gg * NBUF + b
                wait(g, b)
                compute(g, b)
                start(g + NBUF, b)
            return 0

        lax.fori_loop(0, (NCHUNK - NBUF) // NBUF, outer, 0)

        for b in range(NBUF):
            g = NCHUNK - NBUF + b
            wait(g, b)
            compute(g, b)

        pltpu.sync_copy(sv, out_hbm.at[pl.ds(wid * PW, PW)])

    return sc_scores


def _loss_sum(scores_2d):
    def body(x_ref, o_ref):
        o_ref[0, 0] = jnp.sum(-jax.nn.log_sigmoid(x_ref[...]))

    out = pl.pallas_call(
        body,
        out_shape=jax.ShapeDtypeStruct((1, 1), jnp.float32),
        out_specs=pl.BlockSpec(memory_space=pltpu.SMEM),
    )(scores_2d)
    return out[0, 0]


def kernel(u_weight, v_weight, pos_u, pos_v, neg_u, neg_v):
    iu = jnp.concatenate(
        [pos_u.reshape(1, -1), jnp.swapaxes(neg_u, 0, 1)]).astype(jnp.int32)
    iv = jnp.concatenate(
        [pos_v.reshape(1, -1), jnp.swapaxes(neg_v, 0, 1)]).astype(jnp.int32)
    P = iu.size
    shp = (NW, P // (NW * CHUNK), CHUNK)
    iug3 = (_PH * (iu // _PB) + iu % _PH).reshape(shp)
    ivg3 = (_PH * (iv // _PB) + iv % _PH).reshape(shp)
    iuo3 = ((iu % _PB) // _PH * EMB_DIM).reshape(shp)
    ivo3 = ((iv % _PB) // _PH * EMB_DIM).reshape(shp)
    u2 = _pack_rows(jnp.swapaxes(u_weight, 0, 1))
    v2 = _pack_rows(jnp.swapaxes(v_weight, 0, 1))
    scores = _make_sc_scores(P)(u2, v2, iug3, ivg3, iuo3, ivo3)
    return _loss_sum(scores.reshape(P // 128, 128))
